# baseline (speedup 1.0000x reference)
import functools

import jax
import jax.numpy as jnp
from jax import lax
from jax.experimental import pallas as pl
from jax.experimental.pallas import tpu as pltpu
from jax.experimental.pallas import tpu_sc as plsc

N = 10000
E = 320000
F_IN = 128
H = 300
C = 64

CHUNKS = E // 128
W_CH = CHUNKS // 32
EXTRA = CHUNKS - 32 * W_CH
ND = 10240
ROWS_D = ND // 16
ROWS_T = N // 16

_mesh = plsc.VectorSubcoreMesh(core_axis_name="c", subcore_axis_name="s")
_sc_params = pltpu.CompilerParams(use_tc_tiling_on_sc=False)


@functools.partial(
    pl.kernel,
    out_type=jax.ShapeDtypeStruct((2, ND), jnp.float32),
    mesh=_mesh,
    compiler_params=_sc_params,
    scratch_types=[
        pltpu.VMEM((W_CH + 1, 128), jnp.int32),
        pltpu.VMEM((128,), jnp.float32),
        pltpu.VMEM((ROWS_D,), jnp.float32),
        pltpu.VMEM_SHARED((ND,), jnp.float32),
        pltpu.SemaphoreType.DMA,
    ],
)
def _sc_degree(ei_hbm, out_hbm, idx_v, ones_v, zero_v, deg_sh, sem):
    cid = lax.axis_index("c")
    sid = lax.axis_index("s")
    wid = sid * 2 + cid
    one16 = jnp.ones((16,), jnp.float32)
    zero16 = jnp.zeros((16,), jnp.float32)
    for i in range(8):
        ones_v[pl.ds(i * 16, 16)] = one16
    for i in range(ROWS_D // 16):
        zero_v[pl.ds(i * 16, 16)] = zero16
    pltpu.sync_copy(zero_v, deg_sh.at[pl.ds(sid * ROWS_D, ROWS_D)])
    pltpu.sync_copy(ei_hbm.at[1, pl.ds(wid * W_CH, W_CH)], idx_v.at[pl.ds(0, W_CH)])

    @pl.when(wid < EXTRA)
    def _():
        pltpu.sync_copy(ei_hbm.at[1, 32 * W_CH + wid], idx_v.at[W_CH])

    plsc.subcore_barrier()
    for g0 in range(0, W_CH, 16):
        hi = min(g0 + 16, W_CH)
        descs = [
            pltpu.async_copy(ones_v, deg_sh.at[idx_v.at[j]], sem, add=True)
            for j in range(g0, hi)
        ]
        for d in descs:
            d.wait()

    @pl.when(wid < EXTRA)
    def _():
        pltpu.sync_copy(ones_v, deg_sh.at[idx_v.at[W_CH]], add=True)

    plsc.subcore_barrier()
    pltpu.sync_copy(
        deg_sh.at[pl.ds(sid * ROWS_D, ROWS_D)],
        out_hbm.at[cid, pl.ds(sid * ROWS_D, ROWS_D)],
    )


@functools.partial(
    pl.kernel,
    out_type=jax.ShapeDtypeStruct((2, N, C), jnp.float32),
    mesh=_mesh,
    compiler_params=_sc_params,
    scratch_types=[
        pltpu.VMEM((W_CH + 1, 128), jnp.int32),
        pltpu.VMEM((W_CH + 1, 128), jnp.int32),
        pltpu.VMEM((8, 128, C), jnp.float32),
        pltpu.SemaphoreType.DMA,
        pltpu.SemaphoreType.DMA,
        pltpu.VMEM_SHARED((N, C), jnp.float32),
    ],
)
def _sc_scatter(g_hbm, ei_hbm, out_hbm, src_v, dst_v, rows_v, gsem, ssem,
                agg_sh):
    cid = lax.axis_index("c")
    sid = lax.axis_index("s")
    wid = sid * 2 + cid
    pltpu.sync_copy(
        g_hbm.at[pl.ds(sid * ROWS_T, ROWS_T)],
        agg_sh.at[pl.ds(sid * ROWS_T, ROWS_T)],
    )
    pltpu.sync_copy(ei_hbm.at[0, pl.ds(wid * W_CH, W_CH)], src_v.at[pl.ds(0, W_CH)])
    pltpu.sync_copy(ei_hbm.at[1, pl.ds(wid * W_CH, W_CH)], dst_v.at[pl.ds(0, W_CH)])

    @pl.when(wid < EXTRA)
    def _():
        pltpu.sync_copy(ei_hbm.at[0, 32 * W_CH + wid], src_v.at[W_CH])
        pltpu.sync_copy(ei_hbm.at[1, 32 * W_CH + wid], dst_v.at[W_CH])

    plsc.subcore_barrier()

    S = 8
    gd = [None] * S
    sd = [None] * S

    def gather(j, slot):
        return pltpu.async_copy(g_hbm.at[src_v.at[j]], rows_v.at[slot], gsem)

    def scatter(j, slot):
        return pltpu.async_copy(
            rows_v.at[slot], agg_sh.at[dst_v.at[j]], ssem, add=True
        )

    for s in range(S):
        gd[s] = gather(s, s)
    for j in range(W_CH):
        slot = j % S
        gd[slot].wait()
        sd[slot] = scatter(j, slot)
        k = j - (S - 1)
        if k >= 0 and k + S < W_CH:
            sd[k % S].wait()
            gd[k % S] = gather(k + S, k % S)
    for j in range(max(0, W_CH - S), W_CH):
        sd[j % S].wait()

    @pl.when(wid < EXTRA)
    def _():
        pltpu.sync_copy(g_hbm.at[src_v.at[W_CH]], rows_v.at[0])
        pltpu.sync_copy(rows_v.at[0], agg_sh.at[dst_v.at[W_CH]], add=True)

    plsc.subcore_barrier()
    pltpu.sync_copy(
        agg_sh.at[pl.ds(sid * ROWS_T, ROWS_T)],
        out_hbm.at[cid, pl.ds(sid * ROWS_T, ROWS_T)],
    )


_R = 400


def _tc_dense_body(x_ref, w1_ref, b1_ref, w2_ref, deg_ref, g_ref, dinv_ref):
    h = jnp.dot(x_ref[...], w1_ref[...],
                preferred_element_type=jnp.float32,
                precision=lax.Precision.HIGHEST)
    h = jnp.maximum(h + b1_ref[...], 0.0)
    hw = jnp.dot(h, w2_ref[...],
                 preferred_element_type=jnp.float32,
                 precision=lax.Precision.HIGHEST)
    d = deg_ref[...]
    dinv = lax.rsqrt(d[:, 0:1] + d[:, 1:2] + 1.0)
    g_ref[...] = hw * dinv
    dinv_ref[...] = dinv


def _tc_dense(x, W1, b1, W2, degT):
    return pl.pallas_call(
        _tc_dense_body,
        grid=(N // _R,),
        in_specs=[
            pl.BlockSpec((_R, F_IN), lambda i: (i, 0)),
            pl.BlockSpec((F_IN, H), lambda i: (0, 0)),
            pl.BlockSpec((1, H), lambda i: (0, 0)),
            pl.BlockSpec((H, C), lambda i: (0, 0)),
            pl.BlockSpec((_R, 2), lambda i: (i, 0)),
        ],
        out_specs=[
            pl.BlockSpec((_R, C), lambda i: (i, 0)),
            pl.BlockSpec((_R, 1), lambda i: (i, 0)),
        ],
        out_shape=[
            jax.ShapeDtypeStruct((N, C), jnp.float32),
            jax.ShapeDtypeStruct((N, 1), jnp.float32),
        ],
    )(x, W1, b1, W2, degT)


def _tc_softmax_body(s_ref, g_ref, dinv_ref, b2_ref, out_ref):
    s = s_ref[0] + s_ref[1] - g_ref[...]
    v = jnp.maximum(s * dinv_ref[...] + b2_ref[...], 0.0)
    m = jnp.max(v, axis=1, keepdims=True)
    lse = jnp.log(jnp.sum(jnp.exp(v - m), axis=1, keepdims=True))
    out_ref[...] = v - m - lse


def _tc_softmax(s_part, g, dinv, b2):
    return pl.pallas_call(
        _tc_softmax_body,
        grid=(N // _R,),
        in_specs=[
            pl.BlockSpec((2, _R, C), lambda i: (0, i, 0)),
            pl.BlockSpec((_R, C), lambda i: (i, 0)),
            pl.BlockSpec((_R, 1), lambda i: (i, 0)),
            pl.BlockSpec((1, C), lambda i: (0, 0)),
        ],
        out_specs=pl.BlockSpec((_R, C), lambda i: (i, 0)),
        out_shape=jax.ShapeDtypeStruct((N, C), jnp.float32),
    )(s_part, g, dinv, b2)


def kernel(x, edge_index, W1, b1, W2, b2):
    ei3 = edge_index.reshape(2, CHUNKS, 128)
    deg2 = _sc_degree(ei3)
    degT = jnp.transpose(deg2)[:N]
    g, dinv = _tc_dense(x, W1, b1.reshape(1, H), W2, degT)
    s_part = _sc_scatter(g, ei3)
    return _tc_softmax(s_part, g, dinv, b2.reshape(1, C))


# speedup vs baseline: 1.1764x; 1.1764x over previous
import functools

import jax
import jax.numpy as jnp
from jax import lax
from jax.experimental import pallas as pl
from jax.experimental.pallas import tpu as pltpu
from jax.experimental.pallas import tpu_sc as plsc

N = 10000
E = 320000
F_IN = 128
H = 300
C = 64

CHUNKS = E // 128
W_CH = CHUNKS // 32
EXTRA = CHUNKS - 32 * W_CH
ND = 10240
ROWS_D = ND // 16
ROWS_T = N // 16

_mesh = plsc.VectorSubcoreMesh(core_axis_name="c", subcore_axis_name="s")
_sc_params = pltpu.CompilerParams(use_tc_tiling_on_sc=False)


@functools.partial(
    pl.kernel,
    out_type=jax.ShapeDtypeStruct((2, ND), jnp.float32),
    mesh=_mesh,
    compiler_params=_sc_params,
    scratch_types=[
        pltpu.VMEM((W_CH + 1, 128), jnp.int32),
        pltpu.VMEM((128,), jnp.float32),
        pltpu.VMEM((ROWS_D,), jnp.float32),
        pltpu.VMEM_SHARED((ND,), jnp.float32),
        pltpu.SemaphoreType.DMA,
    ],
)
def _sc_degree(ei_hbm, out_hbm, idx_v, ones_v, zero_v, deg_sh, sem):
    cid = lax.axis_index("c")
    sid = lax.axis_index("s")
    wid = sid * 2 + cid
    one16 = jnp.ones((16,), jnp.float32)
    zero16 = jnp.zeros((16,), jnp.float32)
    for i in range(8):
        ones_v[pl.ds(i * 16, 16)] = one16
    for i in range(ROWS_D // 16):
        zero_v[pl.ds(i * 16, 16)] = zero16
    pltpu.sync_copy(zero_v, deg_sh.at[pl.ds(sid * ROWS_D, ROWS_D)])
    pltpu.sync_copy(ei_hbm.at[1, pl.ds(wid * W_CH, W_CH)], idx_v.at[pl.ds(0, W_CH)])

    @pl.when(wid < EXTRA)
    def _():
        pltpu.sync_copy(ei_hbm.at[1, 32 * W_CH + wid], idx_v.at[W_CH])

    plsc.subcore_barrier()
    for g0 in range(0, W_CH, 16):
        hi = min(g0 + 16, W_CH)
        descs = [
            pltpu.async_copy(ones_v, deg_sh.at[idx_v.at[j]], sem, add=True)
            for j in range(g0, hi)
        ]
        for d in descs:
            d.wait()

    @pl.when(wid < EXTRA)
    def _():
        pltpu.sync_copy(ones_v, deg_sh.at[idx_v.at[W_CH]], add=True)

    plsc.subcore_barrier()
    pltpu.sync_copy(
        deg_sh.at[pl.ds(sid * ROWS_D, ROWS_D)],
        out_hbm.at[cid, pl.ds(sid * ROWS_D, ROWS_D)],
    )


@functools.partial(
    pl.kernel,
    out_type=jax.ShapeDtypeStruct((2, N, C), jnp.float32),
    mesh=_mesh,
    compiler_params=_sc_params,
    scratch_types=[
        pltpu.VMEM((W_CH + 1, 128), jnp.int32),
        pltpu.VMEM((W_CH + 1, 128), jnp.int32),
        pltpu.VMEM((8, 128, C), jnp.float32),
        pltpu.SemaphoreType.DMA,
        pltpu.SemaphoreType.DMA,
        pltpu.VMEM_SHARED((N, C), jnp.float32),
    ],
)
def _sc_scatter(g_hbm, ei_hbm, out_hbm, src_v, dst_v, rows_v, gsem, ssem,
                agg_sh):
    cid = lax.axis_index("c")
    sid = lax.axis_index("s")
    wid = sid * 2 + cid
    pltpu.sync_copy(
        g_hbm.at[pl.ds(sid * ROWS_T, ROWS_T)],
        agg_sh.at[pl.ds(sid * ROWS_T, ROWS_T)],
    )
    pltpu.sync_copy(ei_hbm.at[0, pl.ds(wid * W_CH, W_CH)], src_v.at[pl.ds(0, W_CH)])
    pltpu.sync_copy(ei_hbm.at[1, pl.ds(wid * W_CH, W_CH)], dst_v.at[pl.ds(0, W_CH)])

    @pl.when(wid < EXTRA)
    def _():
        pltpu.sync_copy(ei_hbm.at[0, 32 * W_CH + wid], src_v.at[W_CH])
        pltpu.sync_copy(ei_hbm.at[1, 32 * W_CH + wid], dst_v.at[W_CH])

    plsc.subcore_barrier()

    gd = [None] * 8
    sd = [None] * 8

    def gather(j, slot):
        return pltpu.async_copy(g_hbm.at[src_v.at[j]], rows_v.at[slot], gsem)

    def scatter(j, slot):
        return pltpu.async_copy(
            rows_v.at[slot], agg_sh.at[dst_v.at[j]], ssem, add=True
        )

    groups = [list(range(i, min(i + 4, W_CH))) for i in range(0, W_CH, 4)]

    def issue_gathers(gi):
        base = (gi % 2) * 4
        for b, j in enumerate(groups[gi]):
            gd[base + b] = gather(j, base + b)

    issue_gathers(0)
    for gi in range(len(groups)):
        cur = (gi % 2) * 4
        for b, _ in enumerate(groups[gi]):
            gd[cur + b].wait()
        for b, j in enumerate(groups[gi]):
            sd[cur + b] = scatter(j, cur + b)
        if gi + 1 < len(groups):
            oth = ((gi + 1) % 2) * 4
            if gi >= 1:
                for b, _ in enumerate(groups[gi - 1]):
                    sd[oth + b].wait()
            issue_gathers(gi + 1)
    for gi in range(max(0, len(groups) - 2), len(groups)):
        cur = (gi % 2) * 4
        for b, _ in enumerate(groups[gi]):
            sd[cur + b].wait()

    @pl.when(wid < EXTRA)
    def _():
        pltpu.sync_copy(g_hbm.at[src_v.at[W_CH]], rows_v.at[0])
        pltpu.sync_copy(rows_v.at[0], agg_sh.at[dst_v.at[W_CH]], add=True)

    plsc.subcore_barrier()
    pltpu.sync_copy(
        agg_sh.at[pl.ds(sid * ROWS_T, ROWS_T)],
        out_hbm.at[cid, pl.ds(sid * ROWS_T, ROWS_T)],
    )


_R = 400


def _dot(a, b):
    return lax.dot_general(a, b, (((1,), (0,)), ((), ())),
                           preferred_element_type=jnp.float32)


def _split_bf16(a):
    hi = a.astype(jnp.bfloat16)
    lo = (a - hi.astype(jnp.float32)).astype(jnp.bfloat16)
    return hi, lo


def _tc_dense_body(x_ref, w1h_ref, w1l_ref, b1_ref, w2h_ref, w2l_ref,
                   deg_ref, g_ref, dinv_ref):
    xh, xl = _split_bf16(x_ref[...])
    w1h = w1h_ref[...]
    h = _dot(xh, w1h) + _dot(xh, w1l_ref[...]) + _dot(xl, w1h)
    h = jnp.maximum(h + b1_ref[...], 0.0)
    hh, hl = _split_bf16(h)
    w2h = w2h_ref[...]
    hw = _dot(hh, w2h) + _dot(hh, w2l_ref[...]) + _dot(hl, w2h)
    d = deg_ref[...]
    dinv = lax.rsqrt(d[:, 0:1] + d[:, 1:2] + 1.0)
    g_ref[...] = hw * dinv
    dinv_ref[...] = dinv


def _tc_dense(x, W1h, W1l, b1, W2h, W2l, degT):
    return pl.pallas_call(
        _tc_dense_body,
        grid=(N // _R,),
        in_specs=[
            pl.BlockSpec((_R, F_IN), lambda i: (i, 0)),
            pl.BlockSpec((F_IN, H), lambda i: (0, 0)),
            pl.BlockSpec((F_IN, H), lambda i: (0, 0)),
            pl.BlockSpec((1, H), lambda i: (0, 0)),
            pl.BlockSpec((H, C), lambda i: (0, 0)),
            pl.BlockSpec((H, C), lambda i: (0, 0)),
            pl.BlockSpec((_R, 2), lambda i: (i, 0)),
        ],
        out_specs=[
            pl.BlockSpec((_R, C), lambda i: (i, 0)),
            pl.BlockSpec((_R, 1), lambda i: (i, 0)),
        ],
        out_shape=[
            jax.ShapeDtypeStruct((N, C), jnp.float32),
            jax.ShapeDtypeStruct((N, 1), jnp.float32),
        ],
    )(x, W1h, W1l, b1, W2h, W2l, degT)


def _tc_softmax_body(s_ref, g_ref, dinv_ref, b2_ref, out_ref):
    s = s_ref[0] + s_ref[1] - g_ref[...]
    v = jnp.maximum(s * dinv_ref[...] + b2_ref[...], 0.0)
    m = jnp.max(v, axis=1, keepdims=True)
    lse = jnp.log(jnp.sum(jnp.exp(v - m), axis=1, keepdims=True))
    out_ref[...] = v - m - lse


def _tc_softmax(s_part, g, dinv, b2):
    return pl.pallas_call(
        _tc_softmax_body,
        grid=(N // _R,),
        in_specs=[
            pl.BlockSpec((2, _R, C), lambda i: (0, i, 0)),
            pl.BlockSpec((_R, C), lambda i: (i, 0)),
            pl.BlockSpec((_R, 1), lambda i: (i, 0)),
            pl.BlockSpec((1, C), lambda i: (0, 0)),
        ],
        out_specs=pl.BlockSpec((_R, C), lambda i: (i, 0)),
        out_shape=jax.ShapeDtypeStruct((N, C), jnp.float32),
    )(s_part, g, dinv, b2)


def kernel(x, edge_index, W1, b1, W2, b2):
    ei3 = edge_index.reshape(2, CHUNKS, 128)
    deg2 = _sc_degree(ei3)
    degT = jnp.transpose(deg2)[:N]
    W1h = W1.astype(jnp.bfloat16)
    W1l = (W1 - W1h.astype(jnp.float32)).astype(jnp.bfloat16)
    W2h = W2.astype(jnp.bfloat16)
    W2l = (W2 - W2h.astype(jnp.float32)).astype(jnp.bfloat16)
    g, dinv = _tc_dense(x, W1h, W1l, b1.reshape(1, H), W2h, W2l, degT)
    s_part = _sc_scatter(g, ei3)
    return _tc_softmax(s_part, g, dinv, b2.reshape(1, C))


# speedup vs baseline: 1.3267x; 1.1277x over previous
import functools

import jax
import jax.numpy as jnp
from jax import lax
from jax.experimental import pallas as pl
from jax.experimental.pallas import tpu as pltpu
from jax.experimental.pallas import tpu_sc as plsc

N = 10000
E = 320000
F_IN = 128
H = 300
C = 64

CHUNKS = E // 128
W_CH = CHUNKS // 32
EXTRA = CHUNKS - 32 * W_CH
ND = 10240
ROWS_D = ND // 16
ROWS_T = N // 16
GSZ = 4

_mesh = plsc.VectorSubcoreMesh(core_axis_name="c", subcore_axis_name="s")
_sc_params = pltpu.CompilerParams(use_tc_tiling_on_sc=False)


@functools.partial(
    pl.kernel,
    out_type=jax.ShapeDtypeStruct((2, ND), jnp.float32),
    mesh=_mesh,
    compiler_params=_sc_params,
    scratch_types=[
        pltpu.VMEM((W_CH + 1, 128), jnp.int32),
        pltpu.VMEM((128,), jnp.float32),
        pltpu.VMEM((ROWS_D,), jnp.float32),
        pltpu.VMEM_SHARED((ND,), jnp.float32),
        pltpu.SemaphoreType.DMA,
    ],
)
def _sc_degree(ei_hbm, out_hbm, idx_v, ones_v, zero_v, deg_sh, sem):
    cid = lax.axis_index("c")
    sid = lax.axis_index("s")
    wid = sid * 2 + cid
    one16 = jnp.ones((16,), jnp.float32)
    zero16 = jnp.zeros((16,), jnp.float32)
    for i in range(8):
        ones_v[pl.ds(i * 16, 16)] = one16
    for i in range(ROWS_D // 16):
        zero_v[pl.ds(i * 16, 16)] = zero16
    pltpu.sync_copy(zero_v, deg_sh.at[pl.ds(sid * ROWS_D, ROWS_D)])
    pltpu.sync_copy(ei_hbm.at[1, pl.ds(wid * W_CH, W_CH)], idx_v.at[pl.ds(0, W_CH)])

    @pl.when(wid < EXTRA)
    def _():
        pltpu.sync_copy(ei_hbm.at[1, 32 * W_CH + wid], idx_v.at[W_CH])

    plsc.subcore_barrier()
    for g0 in range(0, W_CH, 16):
        hi = min(g0 + 16, W_CH)
        descs = [
            pltpu.async_copy(ones_v, deg_sh.at[idx_v.at[j]], sem, add=True)
            for j in range(g0, hi)
        ]
        for d in descs:
            d.wait()

    @pl.when(wid < EXTRA)
    def _():
        pltpu.sync_copy(ones_v, deg_sh.at[idx_v.at[W_CH]], add=True)

    plsc.subcore_barrier()
    pltpu.sync_copy(
        deg_sh.at[pl.ds(sid * ROWS_D, ROWS_D)],
        out_hbm.at[cid, pl.ds(sid * ROWS_D, ROWS_D)],
    )


@functools.partial(
    pl.kernel,
    out_type=jax.ShapeDtypeStruct((2, N, C), jnp.float32),
    mesh=_mesh,
    compiler_params=_sc_params,
    scratch_types=[
        pltpu.VMEM((W_CH + 1, 128), jnp.int32),
        pltpu.VMEM((W_CH + 1, 128), jnp.int32),
        pltpu.VMEM((2 * GSZ, 128, C), jnp.float32),
        pltpu.SemaphoreType.DMA,
        pltpu.SemaphoreType.DMA,
        pltpu.VMEM_SHARED((N, C), jnp.float32),
    ],
)
def _sc_scatter(g_hbm, ei_hbm, out_hbm, src_v, dst_v, rows_v, gsem, ssem,
                agg_sh):
    cid = lax.axis_index("c")
    sid = lax.axis_index("s")
    wid = sid * 2 + cid
    @pl.when(cid == 0)
    def _():
        pltpu.sync_copy(
            g_hbm.at[pl.ds(sid * ROWS_T, ROWS_T)],
            agg_sh.at[pl.ds(sid * ROWS_T, ROWS_T)],
        )

    @pl.when(cid == 1)
    def _():
        zero16 = jnp.zeros((16,), jnp.float32)

        def zrow(r, carry):
            for c4 in range(C // 16):
                rows_v[0, r, pl.ds(c4 * 16, 16)] = zero16
            return carry

        lax.fori_loop(0, 128, zrow, 0)
        for r0 in range(0, ROWS_T // 128 * 128, 128):
            pltpu.sync_copy(
                rows_v.at[0],
                agg_sh.at[pl.ds(sid * ROWS_T + r0, 128)],
            )
        rem = ROWS_T % 128
        pltpu.sync_copy(
            rows_v.at[0, pl.ds(0, rem)],
            agg_sh.at[pl.ds(sid * ROWS_T + ROWS_T - rem, rem)],
        )
    pltpu.sync_copy(ei_hbm.at[0, pl.ds(wid * W_CH, W_CH)], src_v.at[pl.ds(0, W_CH)])
    pltpu.sync_copy(ei_hbm.at[1, pl.ds(wid * W_CH, W_CH)], dst_v.at[pl.ds(0, W_CH)])

    @pl.when(wid < EXTRA)
    def _():
        pltpu.sync_copy(ei_hbm.at[0, 32 * W_CH + wid], src_v.at[W_CH])
        pltpu.sync_copy(ei_hbm.at[1, 32 * W_CH + wid], dst_v.at[W_CH])

    plsc.subcore_barrier()

    gd = [None] * (2 * GSZ)
    sd = [None] * (2 * GSZ)

    def gather(j, slot):
        return pltpu.async_copy(g_hbm.at[src_v.at[j]], rows_v.at[slot], gsem)

    def scatter(j, slot):
        return pltpu.async_copy(
            rows_v.at[slot], agg_sh.at[dst_v.at[j]], ssem, add=True
        )

    groups = [list(range(i, min(i + GSZ, W_CH))) for i in range(0, W_CH, GSZ)]

    def issue_gathers(gi):
        base = (gi % 2) * GSZ
        for b, j in enumerate(groups[gi]):
            gd[base + b] = gather(j, base + b)

    issue_gathers(0)
    for gi in range(len(groups)):
        cur = (gi % 2) * GSZ
        for b, _ in enumerate(groups[gi]):
            gd[cur + b].wait()
        for b, j in enumerate(groups[gi]):
            sd[cur + b] = scatter(j, cur + b)
        if gi + 1 < len(groups):
            oth = ((gi + 1) % 2) * GSZ
            if gi >= 1:
                for b, _ in enumerate(groups[gi - 1]):
                    sd[oth + b].wait()
            issue_gathers(gi + 1)
    for gi in range(max(0, len(groups) - 2), len(groups)):
        cur = (gi % 2) * GSZ
        for b, _ in enumerate(groups[gi]):
            sd[cur + b].wait()

    @pl.when(wid < EXTRA)
    def _():
        pltpu.sync_copy(g_hbm.at[src_v.at[W_CH]], rows_v.at[0])
        pltpu.sync_copy(rows_v.at[0], agg_sh.at[dst_v.at[W_CH]], add=True)

    plsc.subcore_barrier()
    pltpu.sync_copy(
        agg_sh.at[pl.ds(sid * ROWS_T, ROWS_T)],
        out_hbm.at[cid, pl.ds(sid * ROWS_T, ROWS_T)],
    )


_R = 2000


def _dot(a, b):
    return lax.dot_general(a, b, (((1,), (0,)), ((), ())),
                           preferred_element_type=jnp.float32)


def _split_bf16(a):
    hi = a.astype(jnp.bfloat16)
    lo = (a - hi.astype(jnp.float32)).astype(jnp.bfloat16)
    return hi, lo


def _tc_dense_body(x_ref, w1h_ref, w1l_ref, b1_ref, w2h_ref, w2l_ref,
                   deg_ref, g_ref, dinv_ref):
    xh, xl = _split_bf16(x_ref[...])
    w1h = w1h_ref[...]
    h = _dot(xh, w1h) + _dot(xh, w1l_ref[...]) + _dot(xl, w1h)
    h = jnp.maximum(h + b1_ref[...], 0.0)
    hh, hl = _split_bf16(h)
    w2h = w2h_ref[...]
    hw = _dot(hh, w2h) + _dot(hh, w2l_ref[...]) + _dot(hl, w2h)
    d = deg_ref[...]
    dinv = lax.rsqrt(d[:, 0:1] + d[:, 1:2] + 1.0)
    g_ref[...] = hw * dinv
    dinv_ref[...] = dinv


def _tc_dense(x, W1h, W1l, b1, W2h, W2l, degT):
    return pl.pallas_call(
        _tc_dense_body,
        grid=(N // _R,),
        in_specs=[
            pl.BlockSpec((_R, F_IN), lambda i: (i, 0)),
            pl.BlockSpec((F_IN, H), lambda i: (0, 0)),
            pl.BlockSpec((F_IN, H), lambda i: (0, 0)),
            pl.BlockSpec((1, H), lambda i: (0, 0)),
            pl.BlockSpec((H, C), lambda i: (0, 0)),
            pl.BlockSpec((H, C), lambda i: (0, 0)),
            pl.BlockSpec((_R, 2), lambda i: (i, 0)),
        ],
        out_specs=[
            pl.BlockSpec((_R, C), lambda i: (i, 0)),
            pl.BlockSpec((_R, 1), lambda i: (i, 0)),
        ],
        out_shape=[
            jax.ShapeDtypeStruct((N, C), jnp.float32),
            jax.ShapeDtypeStruct((N, 1), jnp.float32),
        ],
    )(x, W1h, W1l, b1, W2h, W2l, degT)


def _tc_softmax_body(s_ref, dinv_ref, b2_ref, out_ref):
    s = s_ref[0] + s_ref[1]
    v = jnp.maximum(s * dinv_ref[...] + b2_ref[...], 0.0)
    m = jnp.max(v, axis=1, keepdims=True)
    lse = jnp.log(jnp.sum(jnp.exp(v - m), axis=1, keepdims=True))
    out_ref[...] = v - m - lse


def _tc_softmax(s_part, dinv, b2):
    return pl.pallas_call(
        _tc_softmax_body,
        grid=(N // _R,),
        in_specs=[
            pl.BlockSpec((2, _R, C), lambda i: (0, i, 0)),
            pl.BlockSpec((_R, 1), lambda i: (i, 0)),
            pl.BlockSpec((1, C), lambda i: (0, 0)),
        ],
        out_specs=pl.BlockSpec((_R, C), lambda i: (i, 0)),
        out_shape=jax.ShapeDtypeStruct((N, C), jnp.float32),
    )(s_part, dinv, b2)


def kernel(x, edge_index, W1, b1, W2, b2):
    ei3 = edge_index.reshape(2, CHUNKS, 128)
    deg2 = _sc_degree(ei3)
    degT = jnp.transpose(deg2)[:N]
    W1h = W1.astype(jnp.bfloat16)
    W1l = (W1 - W1h.astype(jnp.float32)).astype(jnp.bfloat16)
    W2h = W2.astype(jnp.bfloat16)
    W2l = (W2 - W2h.astype(jnp.float32)).astype(jnp.bfloat16)
    g, dinv = _tc_dense(x, W1h, W1l, b1.reshape(1, H), W2h, W2l, degT)
    s_part = _sc_scatter(g, ei3)
    return _tc_softmax(s_part, dinv, b2.reshape(1, C))


# speedup vs baseline: 1.4356x; 1.0821x over previous
import functools

import jax
import jax.numpy as jnp
from jax import lax
from jax.experimental import pallas as pl
from jax.experimental.pallas import tpu as pltpu
from jax.experimental.pallas import tpu_sc as plsc

N = 10000
E = 320000
F_IN = 128
H = 300
C = 64

CHUNKS = E // 128
W_CH = CHUNKS // 32
EXTRA = CHUNKS - 32 * W_CH
ND = 10240
ROWS_D = ND // 16
ROWS_T = N // 16
GSZ = 4
MULTI = 1

_mesh = plsc.VectorSubcoreMesh(core_axis_name="c", subcore_axis_name="s")
_sc_params = pltpu.CompilerParams(use_tc_tiling_on_sc=False)


@functools.partial(
    pl.kernel,
    out_type=jax.ShapeDtypeStruct((2, ND), jnp.float32),
    mesh=_mesh,
    compiler_params=_sc_params,
    scratch_types=[
        pltpu.VMEM((W_CH + 1, 128), jnp.int32),
        pltpu.VMEM((128,), jnp.float32),
        pltpu.VMEM((ROWS_D,), jnp.float32),
        pltpu.VMEM_SHARED((ND,), jnp.float32),
        pltpu.SemaphoreType.DMA,
    ],
)
def _sc_degree(ei_hbm, out_hbm, idx_v, ones_v, zero_v, deg_sh, sem):
    cid = lax.axis_index("c")
    sid = lax.axis_index("s")
    wid = sid * 2 + cid
    one16 = jnp.ones((16,), jnp.float32)
    zero16 = jnp.zeros((16,), jnp.float32)
    for i in range(8):
        ones_v[pl.ds(i * 16, 16)] = one16
    for i in range(ROWS_D // 16):
        zero_v[pl.ds(i * 16, 16)] = zero16
    pltpu.sync_copy(zero_v, deg_sh.at[pl.ds(sid * ROWS_D, ROWS_D)])
    pltpu.sync_copy(ei_hbm.at[1, pl.ds(wid * W_CH, W_CH)], idx_v.at[pl.ds(0, W_CH)])

    @pl.when(wid < EXTRA)
    def _():
        pltpu.sync_copy(ei_hbm.at[1, 32 * W_CH + wid], idx_v.at[W_CH])

    plsc.subcore_barrier()
    for g0 in range(0, W_CH, 16):
        hi = min(g0 + 16, W_CH)
        descs = [
            pltpu.async_copy(ones_v, deg_sh.at[idx_v.at[j]], sem, add=True)
            for j in range(g0, hi)
        ]
        for d in descs:
            d.wait()

    @pl.when(wid < EXTRA)
    def _():
        pltpu.sync_copy(ones_v, deg_sh.at[idx_v.at[W_CH]], add=True)

    plsc.subcore_barrier()
    pltpu.sync_copy(
        deg_sh.at[pl.ds(sid * ROWS_D, ROWS_D)],
        out_hbm.at[cid, pl.ds(sid * ROWS_D, ROWS_D)],
    )


@functools.partial(
    pl.kernel,
    out_type=jax.ShapeDtypeStruct((2, N, C), jnp.float32),
    mesh=_mesh,
    compiler_params=_sc_params,
    scratch_types=[
        pltpu.VMEM((W_CH + 1, 128), jnp.int32),
        pltpu.VMEM((W_CH + 1, 128), jnp.int32),
        pltpu.VMEM((2 * GSZ, 128, C), jnp.float32),
        pltpu.SemaphoreType.DMA,
        pltpu.SemaphoreType.DMA,
        pltpu.VMEM_SHARED((N, C), jnp.float32),
    ],
)
def _sc_scatter(g_hbm, ei_hbm, out_hbm, src_v, dst_v, rows_v, gsem, ssem,
                agg_sh):
    cid = lax.axis_index("c")
    sid = lax.axis_index("s")
    wid = sid * 2 + cid
    @pl.when(cid == 0)
    def _():
        pltpu.sync_copy(
            g_hbm.at[pl.ds(sid * ROWS_T, ROWS_T)],
            agg_sh.at[pl.ds(sid * ROWS_T, ROWS_T)],
        )

    @pl.when(cid == 1)
    def _():
        zero16 = jnp.zeros((16,), jnp.float32)

        def zrow(r, carry):
            for c4 in range(C // 16):
                rows_v[0, r, pl.ds(c4 * 16, 16)] = zero16
            return carry

        lax.fori_loop(0, 128, zrow, 0)
        for r0 in range(0, ROWS_T // 128 * 128, 128):
            pltpu.sync_copy(
                rows_v.at[0],
                agg_sh.at[pl.ds(sid * ROWS_T + r0, 128)],
            )
        rem = ROWS_T % 128
        pltpu.sync_copy(
            rows_v.at[0, pl.ds(0, rem)],
            agg_sh.at[pl.ds(sid * ROWS_T + ROWS_T - rem, rem)],
        )
    pltpu.sync_copy(ei_hbm.at[0, pl.ds(wid * W_CH, W_CH)], src_v.at[pl.ds(0, W_CH)])
    pltpu.sync_copy(ei_hbm.at[1, pl.ds(wid * W_CH, W_CH)], dst_v.at[pl.ds(0, W_CH)])

    @pl.when(wid < EXTRA)
    def _():
        pltpu.sync_copy(ei_hbm.at[0, 32 * W_CH + wid], src_v.at[W_CH])
        pltpu.sync_copy(ei_hbm.at[1, 32 * W_CH + wid], dst_v.at[W_CH])

    plsc.subcore_barrier()

    gd = [None] * (2 * GSZ)
    sd = [None] * (2 * GSZ)
    NJ = W_CH

    def gather(j, slot):
        return pltpu.async_copy(g_hbm.at[src_v.at[j]], rows_v.at[slot], gsem)

    def scatter(j, slot):
        return pltpu.async_copy(
            rows_v.at[slot], agg_sh.at[dst_v.at[j]], ssem, add=True
        )

    groups = [list(range(i, min(i + GSZ, NJ))) for i in range(0, NJ, GSZ)]

    def issue_gathers(gi):
        base = (gi % 2) * GSZ
        for b, j in enumerate(groups[gi]):
            gd[base + b] = gather(j, base + b)

    issue_gathers(0)
    for gi in range(len(groups)):
        cur = (gi % 2) * GSZ
        for b, _ in enumerate(groups[gi]):
            gd[cur + b].wait()
        for b, j in enumerate(groups[gi]):
            sd[cur + b] = scatter(j, cur + b)
        if gi + 1 < len(groups):
            oth = ((gi + 1) % 2) * GSZ
            if gi >= 1:
                for b, _ in enumerate(groups[gi - 1]):
                    sd[oth + b].wait()
            issue_gathers(gi + 1)
    for gi in range(max(0, len(groups) - 2), len(groups)):
        cur = (gi % 2) * GSZ
        for b, _ in enumerate(groups[gi]):
            sd[cur + b].wait()

    @pl.when(wid < EXTRA)
    def _():
        pltpu.sync_copy(g_hbm.at[src_v.at[W_CH]], rows_v.at[0])
        pltpu.sync_copy(rows_v.at[0], agg_sh.at[dst_v.at[W_CH]], add=True)

    plsc.subcore_barrier()
    pltpu.sync_copy(
        agg_sh.at[pl.ds(sid * ROWS_T, ROWS_T)],
        out_hbm.at[cid, pl.ds(sid * ROWS_T, ROWS_T)],
    )


_R = 2000


def _dot(a, b):
    return lax.dot_general(a, b, (((1,), (0,)), ((), ())),
                           preferred_element_type=jnp.float32)


HP = 384


def _tc_dense_body(x_ref, w1a_ref, w1b_ref, b1_ref, w2c_ref, deg_ref,
                   g_ref, dinv_ref):
    x = x_ref[...]
    xh = x.astype(jnp.bfloat16)
    xl = (x - xh.astype(jnp.float32)).astype(jnp.bfloat16)
    xcat = jnp.concatenate([xh, xl], axis=1)
    t = _dot(xcat, w1a_ref[...]) + _dot(xh, w1b_ref[...])
    h = jnp.maximum(t + b1_ref[...], 0.0)
    hh = h.astype(jnp.bfloat16)
    hl = (h - hh.astype(jnp.float32)).astype(jnp.bfloat16)
    hcat = jnp.concatenate([hh, hl], axis=1)
    u = _dot(hcat, w2c_ref[...])
    hw = u[:, :C] + u[:, C:]
    d = deg_ref[...]
    dinv = lax.rsqrt(d[:, 0:1] + d[:, 1:2] + 1.0)
    g_ref[...] = hw * dinv
    dinv_ref[...] = dinv


def _tc_dense(x, W1a, W1b, b1p, W2c, degT):
    return pl.pallas_call(
        _tc_dense_body,
        grid=(N // _R,),
        in_specs=[
            pl.BlockSpec((_R, F_IN), lambda i: (i, 0)),
            pl.BlockSpec((2 * F_IN, HP), lambda i: (0, 0)),
            pl.BlockSpec((F_IN, HP), lambda i: (0, 0)),
            pl.BlockSpec((1, HP), lambda i: (0, 0)),
            pl.BlockSpec((2 * HP, 2 * C), lambda i: (0, 0)),
            pl.BlockSpec((_R, 2), lambda i: (i, 0)),
        ],
        out_specs=[
            pl.BlockSpec((_R, C), lambda i: (i, 0)),
            pl.BlockSpec((_R, 1), lambda i: (i, 0)),
        ],
        out_shape=[
            jax.ShapeDtypeStruct((N, C), jnp.float32),
            jax.ShapeDtypeStruct((N, 1), jnp.float32),
        ],
    )(x, W1a, W1b, b1p, W2c, degT)


def _tc_softmax_body(s_ref, dinv_ref, b2_ref, out_ref):
    s = s_ref[0] + s_ref[1]
    v = jnp.maximum(s * dinv_ref[...] + b2_ref[...], 0.0)
    m = jnp.max(v, axis=1, keepdims=True)
    lse = jnp.log(jnp.sum(jnp.exp(v - m), axis=1, keepdims=True))
    out_ref[...] = v - m - lse


def _tc_softmax(s_part, dinv, b2):
    return pl.pallas_call(
        _tc_softmax_body,
        grid=(N // _R,),
        in_specs=[
            pl.BlockSpec((2, _R, C), lambda i: (0, i, 0)),
            pl.BlockSpec((_R, 1), lambda i: (i, 0)),
            pl.BlockSpec((1, C), lambda i: (0, 0)),
        ],
        out_specs=pl.BlockSpec((_R, C), lambda i: (i, 0)),
        out_shape=jax.ShapeDtypeStruct((N, C), jnp.float32),
    )(s_part, dinv, b2)


def kernel(x, edge_index, W1, b1, W2, b2):
    ei3 = edge_index.reshape(2, CHUNKS, 128)
    deg2 = _sc_degree(ei3)
    degT = jnp.transpose(deg2)[:N]
    bf = jnp.bfloat16
    W1h = W1.astype(bf)
    W1l = (W1 - W1h.astype(jnp.float32)).astype(bf)
    W2h = W2.astype(bf)
    W2l = (W2 - W2h.astype(jnp.float32)).astype(bf)
    zc = jnp.zeros((F_IN, HP - H), bf)
    W1hp = jnp.concatenate([W1h, zc], 1)
    W1a = jnp.concatenate([W1hp, W1hp], 0)
    W1b = jnp.concatenate([W1l, zc], 1)
    b1p = jnp.pad(b1, (0, HP - H)).reshape(1, HP)
    z64 = jnp.zeros((H, C), bf)
    zrow = jnp.zeros((HP - H, 2 * C), bf)
    W2c = jnp.concatenate([
        jnp.concatenate([W2h, W2l], 1), zrow,
        jnp.concatenate([W2h, z64], 1), zrow,
    ], 0)
    g, dinv = _tc_dense(x, W1a, W1b, b1p, W2c, degT)
    s_part = _sc_scatter(g, ei3)
    return _tc_softmax(s_part, dinv, b2.reshape(1, C))


# speedup vs baseline: 1.5340x; 1.0685x over previous
import functools

import jax
import jax.numpy as jnp
from jax import lax
from jax.experimental import pallas as pl
from jax.experimental.pallas import tpu as pltpu
from jax.experimental.pallas import tpu_sc as plsc

N = 10000
E = 320000
F_IN = 128
H = 300
C = 64

CHUNKS = E // 128
W_CH = CHUNKS // 32
EXTRA = CHUNKS - 32 * W_CH
ND = 10240
ROWS_D = ND // 16
ROWS_T = N // 16
GSZ = 4
MULTI = 1

_mesh = plsc.VectorSubcoreMesh(core_axis_name="c", subcore_axis_name="s")
_sc_params = pltpu.CompilerParams(use_tc_tiling_on_sc=False)


@functools.partial(
    pl.kernel,
    out_type=jax.ShapeDtypeStruct((2, ND), jnp.float32),
    mesh=_mesh,
    compiler_params=_sc_params,
    scratch_types=[
        pltpu.VMEM((W_CH + 1, 128), jnp.int32),
        pltpu.VMEM((128,), jnp.float32),
        pltpu.VMEM((ROWS_D,), jnp.float32),
        pltpu.VMEM_SHARED((ND,), jnp.float32),
        pltpu.SemaphoreType.DMA,
    ],
)
def _sc_degree(ei_hbm, out_hbm, idx_v, ones_v, zero_v, deg_sh, sem):
    cid = lax.axis_index("c")
    sid = lax.axis_index("s")
    wid = sid * 2 + cid
    one16 = jnp.ones((16,), jnp.float32)
    zero16 = jnp.zeros((16,), jnp.float32)
    for i in range(8):
        ones_v[pl.ds(i * 16, 16)] = one16
    for i in range(ROWS_D // 16):
        zero_v[pl.ds(i * 16, 16)] = zero16
    pltpu.sync_copy(zero_v, deg_sh.at[pl.ds(sid * ROWS_D, ROWS_D)])
    pltpu.sync_copy(ei_hbm.at[1, pl.ds(wid * W_CH, W_CH)], idx_v.at[pl.ds(0, W_CH)])

    @pl.when(wid < EXTRA)
    def _():
        pltpu.sync_copy(ei_hbm.at[1, 32 * W_CH + wid], idx_v.at[W_CH])

    plsc.subcore_barrier()
    for g0 in range(0, W_CH, 16):
        hi = min(g0 + 16, W_CH)
        descs = [
            pltpu.async_copy(ones_v, deg_sh.at[idx_v.at[j]], sem, add=True)
            for j in range(g0, hi)
        ]
        for d in descs:
            d.wait()

    @pl.when(wid < EXTRA)
    def _():
        pltpu.sync_copy(ones_v, deg_sh.at[idx_v.at[W_CH]], add=True)

    plsc.subcore_barrier()
    pltpu.sync_copy(
        deg_sh.at[pl.ds(sid * ROWS_D, ROWS_D)],
        out_hbm.at[cid, pl.ds(sid * ROWS_D, ROWS_D)],
    )


@functools.partial(
    pl.kernel,
    out_type=jax.ShapeDtypeStruct((2, N, C), jnp.float32),
    mesh=_mesh,
    compiler_params=_sc_params,
    scratch_types=[
        pltpu.VMEM((W_CH + 1, 128), jnp.int32),
        pltpu.VMEM((W_CH + 1, 128), jnp.int32),
        pltpu.VMEM((2 * GSZ, 128, C), jnp.float32),
        pltpu.SemaphoreType.DMA,
        pltpu.SemaphoreType.DMA,
        pltpu.VMEM_SHARED((N, C), jnp.float32),
    ],
)
def _sc_scatter(g_hbm, ei_hbm, out_hbm, src_v, dst_v, rows_v, gsem, ssem,
                agg_sh):
    cid = lax.axis_index("c")
    sid = lax.axis_index("s")
    wid = sid * 2 + cid
    @pl.when(cid == 0)
    def _():
        pltpu.sync_copy(
            g_hbm.at[pl.ds(sid * ROWS_T, ROWS_T)],
            agg_sh.at[pl.ds(sid * ROWS_T, ROWS_T)],
        )

    @pl.when(cid == 1)
    def _():
        zero16 = jnp.zeros((16,), jnp.float32)

        def zrow(r, carry):
            for c4 in range(C // 16):
                rows_v[0, r, pl.ds(c4 * 16, 16)] = zero16
            return carry

        lax.fori_loop(0, 128, zrow, 0)
        for r0 in range(0, ROWS_T // 128 * 128, 128):
            pltpu.sync_copy(
                rows_v.at[0],
                agg_sh.at[pl.ds(sid * ROWS_T + r0, 128)],
            )
        rem = ROWS_T % 128
        pltpu.sync_copy(
            rows_v.at[0, pl.ds(0, rem)],
            agg_sh.at[pl.ds(sid * ROWS_T + ROWS_T - rem, rem)],
        )
    pltpu.sync_copy(ei_hbm.at[0, pl.ds(wid * W_CH, W_CH)], src_v.at[pl.ds(0, W_CH)])
    pltpu.sync_copy(ei_hbm.at[1, pl.ds(wid * W_CH, W_CH)], dst_v.at[pl.ds(0, W_CH)])

    @pl.when(wid < EXTRA)
    def _():
        pltpu.sync_copy(ei_hbm.at[0, 32 * W_CH + wid], src_v.at[W_CH])
        pltpu.sync_copy(ei_hbm.at[1, 32 * W_CH + wid], dst_v.at[W_CH])

    plsc.subcore_barrier()

    gd = [None] * (2 * GSZ)
    sd = [None] * (2 * GSZ)
    NJ = W_CH

    def gather(j, slot):
        return pltpu.async_copy(g_hbm.at[src_v.at[j]], rows_v.at[slot], gsem)

    def scatter(j, slot):
        return pltpu.async_copy(
            rows_v.at[slot], agg_sh.at[dst_v.at[j]], ssem, add=True
        )

    groups = [list(range(i, min(i + GSZ, NJ))) for i in range(0, NJ, GSZ)]

    def issue_gathers(gi):
        base = (gi % 2) * GSZ
        for b, j in enumerate(groups[gi]):
            gd[base + b] = gather(j, base + b)

    issue_gathers(0)
    for gi in range(len(groups)):
        cur = (gi % 2) * GSZ
        for b, _ in enumerate(groups[gi]):
            gd[cur + b].wait()
        for b, j in enumerate(groups[gi]):
            sd[cur + b] = scatter(j, cur + b)
        if gi + 1 < len(groups):
            oth = ((gi + 1) % 2) * GSZ
            if gi >= 1:
                for b, _ in enumerate(groups[gi - 1]):
                    sd[oth + b].wait()
            issue_gathers(gi + 1)
    for gi in range(max(0, len(groups) - 2), len(groups)):
        cur = (gi % 2) * GSZ
        for b, _ in enumerate(groups[gi]):
            sd[cur + b].wait()

    @pl.when(wid < EXTRA)
    def _():
        pltpu.sync_copy(g_hbm.at[src_v.at[W_CH]], rows_v.at[0])
        pltpu.sync_copy(rows_v.at[0], agg_sh.at[dst_v.at[W_CH]], add=True)

    plsc.subcore_barrier()
    pltpu.sync_copy(
        agg_sh.at[pl.ds(sid * ROWS_T, ROWS_T)],
        out_hbm.at[cid, pl.ds(sid * ROWS_T, ROWS_T)],
    )


_R = 2000


def _dot(a, b):
    return lax.dot_general(a, b, (((1,), (0,)), ((), ())),
                           preferred_element_type=jnp.float32)


HP = 384


def _tc_dense_body(x_ref, w1a_ref, w1b_ref, b1_ref, w2c_ref, deg_ref,
                   g_ref):
    x = x_ref[...]
    xh = x.astype(jnp.bfloat16)
    xl = (x - xh.astype(jnp.float32)).astype(jnp.bfloat16)
    xcat = jnp.concatenate([xh, xl], axis=1)
    t = _dot(xcat, w1a_ref[...]) + _dot(xh, w1b_ref[...])
    h = jnp.maximum(t + b1_ref[...], 0.0)
    hh = h.astype(jnp.bfloat16)
    hl = (h - hh.astype(jnp.float32)).astype(jnp.bfloat16)
    hcat = jnp.concatenate([hh, hl], axis=1)
    u = _dot(hcat, w2c_ref[...])
    hw = u[:, :C] + u[:, C:]
    d = deg_ref[...]
    dinv = lax.rsqrt(d[:, 0:1] + d[:, 1:2] + 1.0)
    g_ref[...] = hw * dinv


def _tc_dense(x, W1a, W1b, b1p, W2c, degT):
    return pl.pallas_call(
        _tc_dense_body,
        grid=(N // _R,),
        in_specs=[
            pl.BlockSpec((_R, F_IN), lambda i: (i, 0)),
            pl.BlockSpec((2 * F_IN, HP), lambda i: (0, 0)),
            pl.BlockSpec((F_IN, HP), lambda i: (0, 0)),
            pl.BlockSpec((1, HP), lambda i: (0, 0)),
            pl.BlockSpec((2 * HP, 2 * C), lambda i: (0, 0)),
            pl.BlockSpec((_R, 2), lambda i: (i, 0)),
        ],
        out_specs=pl.BlockSpec((_R, C), lambda i: (i, 0)),
        out_shape=jax.ShapeDtypeStruct((N, C), jnp.float32),
    )(x, W1a, W1b, b1p, W2c, degT)


_R2 = _R // 2


def _tc_softmax_body(s_ref, deg_ref, b2_ref, out_ref):
    sp = s_ref[...]
    s = sp[0] + sp[1]
    d = deg_ref[...]
    di_e = lax.rsqrt(d[:, 0:1] + d[:, 1:2] + 1.0)
    di_o = lax.rsqrt(d[:, 2:3] + d[:, 3:4] + 1.0)
    dcat = jnp.concatenate(
        [jnp.broadcast_to(di_e, (_R2, C)), jnp.broadcast_to(di_o, (_R2, C))],
        axis=1)
    v = jnp.maximum(s * dcat + b2_ref[...], 0.0)
    vl = v[:, :C]
    vh = v[:, C:]
    ml = jnp.max(vl, axis=1, keepdims=True)
    mh = jnp.max(vh, axis=1, keepdims=True)
    ll = jnp.log(jnp.sum(jnp.exp(vl - ml), axis=1, keepdims=True))
    lh = jnp.log(jnp.sum(jnp.exp(vh - mh), axis=1, keepdims=True))
    out_ref[...] = jnp.concatenate([vl - ml - ll, vh - mh - lh], axis=1)


def _tc_softmax(s2, degP, b2c):
    return pl.pallas_call(
        _tc_softmax_body,
        grid=(N // _R,),
        in_specs=[
            pl.BlockSpec((2, _R2, 2 * C), lambda i: (0, i, 0)),
            pl.BlockSpec((_R2, 4), lambda i: (i, 0)),
            pl.BlockSpec((1, 2 * C), lambda i: (0, 0)),
        ],
        out_specs=pl.BlockSpec((_R2, 2 * C), lambda i: (i, 0)),
        out_shape=jax.ShapeDtypeStruct((N // 2, 2 * C), jnp.float32),
    )(s2, degP, b2c)


def kernel(x, edge_index, W1, b1, W2, b2):
    ei3 = edge_index.reshape(2, CHUNKS, 128)
    deg2 = _sc_degree(ei3)
    degT = jnp.transpose(deg2)[:N]
    bf = jnp.bfloat16
    W1h = W1.astype(bf)
    W1l = (W1 - W1h.astype(jnp.float32)).astype(bf)
    W2h = W2.astype(bf)
    W2l = (W2 - W2h.astype(jnp.float32)).astype(bf)
    zc = jnp.zeros((F_IN, HP - H), bf)
    W1hp = jnp.concatenate([W1h, zc], 1)
    W1a = jnp.concatenate([W1hp, W1hp], 0)
    W1b = jnp.concatenate([W1l, zc], 1)
    b1p = jnp.pad(b1, (0, HP - H)).reshape(1, HP)
    z64 = jnp.zeros((H, C), bf)
    zrow = jnp.zeros((HP - H, 2 * C), bf)
    W2c = jnp.concatenate([
        jnp.concatenate([W2h, W2l], 1), zrow,
        jnp.concatenate([W2h, z64], 1), zrow,
    ], 0)
    g = _tc_dense(x, W1a, W1b, b1p, W2c, degT)
    s_part = _sc_scatter(g, ei3)
    s2 = s_part.reshape(2, N // 2, 2 * C)
    degP = jnp.transpose(deg2)[:N].reshape(N // 2, 4)
    b2c = jnp.concatenate([b2, b2]).reshape(1, 2 * C)
    out2 = _tc_softmax(s2, degP, b2c)
    return out2.reshape(N, C)


# speedup vs baseline: 1.5431x; 1.0059x over previous
import functools

import jax
import jax.numpy as jnp
from jax import lax
from jax.experimental import pallas as pl
from jax.experimental.pallas import tpu as pltpu
from jax.experimental.pallas import tpu_sc as plsc

N = 10000
E = 320000
F_IN = 128
H = 300
C = 64

CHUNKS = E // 128
W_CH = CHUNKS // 32
EXTRA = CHUNKS - 32 * W_CH
ND = 10240
ROWS_D = ND // 16
ROWS_T = N // 16
GSZ = 4
MULTI = 1

_mesh = plsc.VectorSubcoreMesh(core_axis_name="c", subcore_axis_name="s")
_sc_params = pltpu.CompilerParams(use_tc_tiling_on_sc=False)


@functools.partial(
    pl.kernel,
    out_type=jax.ShapeDtypeStruct((2, ND), jnp.float32),
    mesh=_mesh,
    compiler_params=_sc_params,
    scratch_types=[
        pltpu.VMEM((W_CH + 1, 128), jnp.int32),
        pltpu.VMEM((128,), jnp.float32),
        pltpu.VMEM((ROWS_D,), jnp.float32),
        pltpu.VMEM_SHARED((ND,), jnp.float32),
        pltpu.SemaphoreType.DMA,
    ],
)
def _sc_degree(ei_hbm, out_hbm, idx_v, ones_v, zero_v, deg_sh, sem):
    cid = lax.axis_index("c")
    sid = lax.axis_index("s")
    wid = sid * 2 + cid
    one16 = jnp.ones((16,), jnp.float32)
    zero16 = jnp.zeros((16,), jnp.float32)
    for i in range(8):
        ones_v[pl.ds(i * 16, 16)] = one16
    for i in range(ROWS_D // 16):
        zero_v[pl.ds(i * 16, 16)] = zero16
    pltpu.sync_copy(zero_v, deg_sh.at[pl.ds(sid * ROWS_D, ROWS_D)])
    pltpu.sync_copy(ei_hbm.at[1, pl.ds(wid * W_CH, W_CH)], idx_v.at[pl.ds(0, W_CH)])

    @pl.when(wid < EXTRA)
    def _():
        pltpu.sync_copy(ei_hbm.at[1, 32 * W_CH + wid], idx_v.at[W_CH])

    plsc.subcore_barrier()
    descs = [
        pltpu.async_copy(ones_v, deg_sh.at[idx_v.at[j]], sem, add=True)
        for j in range(W_CH)
    ]
    for d in descs:
        d.wait()

    @pl.when(wid < EXTRA)
    def _():
        pltpu.sync_copy(ones_v, deg_sh.at[idx_v.at[W_CH]], add=True)

    plsc.subcore_barrier()
    pltpu.sync_copy(
        deg_sh.at[pl.ds(sid * ROWS_D, ROWS_D)],
        out_hbm.at[cid, pl.ds(sid * ROWS_D, ROWS_D)],
    )


@functools.partial(
    pl.kernel,
    out_type=jax.ShapeDtypeStruct((2, N, C), jnp.float32),
    mesh=_mesh,
    compiler_params=_sc_params,
    scratch_types=[
        pltpu.VMEM((W_CH + 1, 128), jnp.int32),
        pltpu.VMEM((W_CH + 1, 128), jnp.int32),
        pltpu.VMEM((2 * GSZ, 128, C), jnp.float32),
        pltpu.SemaphoreType.DMA,
        pltpu.SemaphoreType.DMA,
        pltpu.VMEM_SHARED((N, C), jnp.float32),
    ],
)
def _sc_scatter(g_hbm, ei_hbm, out_hbm, src_v, dst_v, rows_v, gsem, ssem,
                agg_sh):
    cid = lax.axis_index("c")
    sid = lax.axis_index("s")
    wid = sid * 2 + cid
    pltpu.sync_copy(ei_hbm.at[0, pl.ds(wid * W_CH, W_CH)], src_v.at[pl.ds(0, W_CH)])
    pltpu.sync_copy(ei_hbm.at[1, pl.ds(wid * W_CH, W_CH)], dst_v.at[pl.ds(0, W_CH)])

    @pl.when(wid < EXTRA)
    def _():
        pltpu.sync_copy(ei_hbm.at[0, 32 * W_CH + wid], src_v.at[W_CH])
        pltpu.sync_copy(ei_hbm.at[1, 32 * W_CH + wid], dst_v.at[W_CH])

    @pl.when(cid == 0)
    def _():
        pltpu.sync_copy(
            g_hbm.at[pl.ds(sid * ROWS_T, ROWS_T)],
            agg_sh.at[pl.ds(sid * ROWS_T, ROWS_T)],
        )

    @pl.when(cid == 1)
    def _():
        zero16 = jnp.zeros((16,), jnp.float32)

        def zrow(r, carry):
            for c4 in range(C // 16):
                rows_v[0, r, pl.ds(c4 * 16, 16)] = zero16
            return carry

        lax.fori_loop(0, 128, zrow, 0)
        for r0 in range(0, ROWS_T // 128 * 128, 128):
            pltpu.sync_copy(
                rows_v.at[0],
                agg_sh.at[pl.ds(sid * ROWS_T + r0, 128)],
            )
        rem = ROWS_T % 128
        pltpu.sync_copy(
            rows_v.at[0, pl.ds(0, rem)],
            agg_sh.at[pl.ds(sid * ROWS_T + ROWS_T - rem, rem)],
        )
    gd = [None] * (2 * GSZ)
    sd = [None] * (2 * GSZ)
    NJ = W_CH

    def gather(j, slot):
        return pltpu.async_copy(g_hbm.at[src_v.at[j]], rows_v.at[slot], gsem)

    def scatter(j, slot):
        return pltpu.async_copy(
            rows_v.at[slot], agg_sh.at[dst_v.at[j]], ssem, add=True
        )

    groups = [list(range(i, min(i + GSZ, NJ))) for i in range(0, NJ, GSZ)]

    def issue_gathers(gi):
        base = (gi % 2) * GSZ
        for b, j in enumerate(groups[gi]):
            gd[base + b] = gather(j, base + b)

    issue_gathers(0)
    plsc.subcore_barrier()
    for gi in range(len(groups)):
        cur = (gi % 2) * GSZ
        for b, _ in enumerate(groups[gi]):
            gd[cur + b].wait()
        for b, j in enumerate(groups[gi]):
            sd[cur + b] = scatter(j, cur + b)
        if gi + 1 < len(groups):
            oth = ((gi + 1) % 2) * GSZ
            if gi >= 1:
                for b, _ in enumerate(groups[gi - 1]):
                    sd[oth + b].wait()
            issue_gathers(gi + 1)
    for gi in range(max(0, len(groups) - 2), len(groups)):
        cur = (gi % 2) * GSZ
        for b, _ in enumerate(groups[gi]):
            sd[cur + b].wait()

    @pl.when(wid < EXTRA)
    def _():
        pltpu.sync_copy(g_hbm.at[src_v.at[W_CH]], rows_v.at[0])
        pltpu.sync_copy(rows_v.at[0], agg_sh.at[dst_v.at[W_CH]], add=True)

    plsc.subcore_barrier()
    pltpu.sync_copy(
        agg_sh.at[pl.ds(sid * ROWS_T, ROWS_T)],
        out_hbm.at[cid, pl.ds(sid * ROWS_T, ROWS_T)],
    )


_R = 2000


def _dot(a, b):
    return lax.dot_general(a, b, (((1,), (0,)), ((), ())),
                           preferred_element_type=jnp.float32)


HP = 384


def _tc_dense_body(x_ref, w1a_ref, w1b_ref, b1_ref, w2c_ref, deg_ref,
                   g_ref):
    x = x_ref[...]
    xh = x.astype(jnp.bfloat16)
    xl = (x - xh.astype(jnp.float32)).astype(jnp.bfloat16)
    xcat = jnp.concatenate([xh, xl], axis=1)
    t = _dot(xcat, w1a_ref[...]) + _dot(xh, w1b_ref[...])
    h = jnp.maximum(t + b1_ref[...], 0.0)
    hh = h.astype(jnp.bfloat16)
    hl = (h - hh.astype(jnp.float32)).astype(jnp.bfloat16)
    hcat = jnp.concatenate([hh, hl], axis=1)
    u = _dot(hcat, w2c_ref[...])
    hw = u[:, :C] + u[:, C:]
    d = deg_ref[...]
    dinv = lax.rsqrt(d[:, 0:1] + d[:, 1:2] + 1.0)
    g_ref[...] = hw * dinv


def _tc_dense(x, W1a, W1b, b1p, W2c, degT):
    return pl.pallas_call(
        _tc_dense_body,
        grid=(N // _R,),
        in_specs=[
            pl.BlockSpec((_R, F_IN), lambda i: (i, 0)),
            pl.BlockSpec((2 * F_IN, HP), lambda i: (0, 0)),
            pl.BlockSpec((F_IN, HP), lambda i: (0, 0)),
            pl.BlockSpec((1, HP), lambda i: (0, 0)),
            pl.BlockSpec((2 * HP, 2 * C), lambda i: (0, 0)),
            pl.BlockSpec((_R, 2), lambda i: (i, 0)),
        ],
        out_specs=pl.BlockSpec((_R, C), lambda i: (i, 0)),
        out_shape=jax.ShapeDtypeStruct((N, C), jnp.float32),
    )(x, W1a, W1b, b1p, W2c, degT)


_R2 = _R // 2


def _tc_softmax_body(s_ref, deg_ref, b2_ref, out_ref):
    sp = s_ref[...]
    s = sp[0] + sp[1]
    d = deg_ref[...]
    di_e = lax.rsqrt(d[:, 0:1] + d[:, 1:2] + 1.0)
    di_o = lax.rsqrt(d[:, 2:3] + d[:, 3:4] + 1.0)
    dcat = jnp.concatenate(
        [jnp.broadcast_to(di_e, (_R2, C)), jnp.broadcast_to(di_o, (_R2, C))],
        axis=1)
    v = jnp.maximum(s * dcat + b2_ref[...], 0.0)
    vl = v[:, :C]
    vh = v[:, C:]
    ml = jnp.max(vl, axis=1, keepdims=True)
    mh = jnp.max(vh, axis=1, keepdims=True)
    ll = jnp.log(jnp.sum(jnp.exp(vl - ml), axis=1, keepdims=True))
    lh = jnp.log(jnp.sum(jnp.exp(vh - mh), axis=1, keepdims=True))
    out_ref[...] = jnp.concatenate([vl - ml - ll, vh - mh - lh], axis=1)


def _tc_softmax(s2, degP, b2c):
    return pl.pallas_call(
        _tc_softmax_body,
        grid=(N // _R,),
        in_specs=[
            pl.BlockSpec((2, _R2, 2 * C), lambda i: (0, i, 0)),
            pl.BlockSpec((_R2, 4), lambda i: (i, 0)),
            pl.BlockSpec((1, 2 * C), lambda i: (0, 0)),
        ],
        out_specs=pl.BlockSpec((_R2, 2 * C), lambda i: (i, 0)),
        out_shape=jax.ShapeDtypeStruct((N // 2, 2 * C), jnp.float32),
    )(s2, degP, b2c)


def kernel(x, edge_index, W1, b1, W2, b2):
    ei3 = edge_index.reshape(2, CHUNKS, 128)
    deg2 = _sc_degree(ei3)
    degT = jnp.transpose(deg2)[:N]
    bf = jnp.bfloat16
    W1h = W1.astype(bf)
    W1l = (W1 - W1h.astype(jnp.float32)).astype(bf)
    W2h = W2.astype(bf)
    W2l = (W2 - W2h.astype(jnp.float32)).astype(bf)
    zc = jnp.zeros((F_IN, HP - H), bf)
    W1hp = jnp.concatenate([W1h, zc], 1)
    W1a = jnp.concatenate([W1hp, W1hp], 0)
    W1b = jnp.concatenate([W1l, zc], 1)
    b1p = jnp.pad(b1, (0, HP - H)).reshape(1, HP)
    z64 = jnp.zeros((H, C), bf)
    zrow = jnp.zeros((HP - H, 2 * C), bf)
    W2c = jnp.concatenate([
        jnp.concatenate([W2h, W2l], 1), zrow,
        jnp.concatenate([W2h, z64], 1), zrow,
    ], 0)
    g = _tc_dense(x, W1a, W1b, b1p, W2c, degT)
    s_part = _sc_scatter(g, ei3)
    s2 = s_part.reshape(2, N // 2, 2 * C)
    degP = jnp.transpose(deg2)[:N].reshape(N // 2, 4)
    b2c = jnp.concatenate([b2, b2]).reshape(1, 2 * C)
    out2 = _tc_softmax(s2, degP, b2c)
    return out2.reshape(N, C)
